# baseline (speedup 1.0000x reference)
import functools

import jax
import jax.numpy as jnp
from jax import lax
from jax.experimental import pallas as pl
from jax.experimental.pallas import tpu as pltpu
from jax.experimental.pallas import tpu_sc as plsc

N = 10000
E = 320000
CHUNK = 128
NCH = E // CHUNK
NW = 32
RPT = N // 16
NP = 10112
RPT_P = NP // 16

_MESH = dict(core_axis_name="c", subcore_axis_name="s",
             num_cores=2, num_subcores=16)


def _fill_vmem_2d(ref, nrows, ncols, value):
    v = jnp.full((16,), value, jnp.float32)

    def body(r, _):
        for j in range(ncols // 16):
            ref[r, pl.ds(j * 16, 16)] = v
        return 0

    lax.fori_loop(0, nrows, body, 0)


def _copy_rows(src_ref, dst_ref, r0, rpt):
    for p in range(rpt // CHUNK):
        pltpu.sync_copy(src_ref, dst_ref.at[pl.ds(r0 + p * CHUNK, CHUNK), :])
    rem = rpt % CHUNK
    if rem:
        pltpu.sync_copy(
            src_ref.at[pl.ds(0, rem), :],
            dst_ref.at[pl.ds(r0 + (rpt // CHUNK) * CHUNK, rem), :],
        )


def _nch_for(wid):
    return (NCH - wid + NW - 1) // NW


def _sc_degree(dst_flat):
    nbuf = 4
    slag = 2

    @functools.partial(
        pl.kernel,
        out_type=jax.ShapeDtypeStruct((2, N, 8), jnp.float32),
        mesh=plsc.VectorSubcoreMesh(**_MESH),
        compiler_params=pltpu.CompilerParams(use_tc_tiling_on_sc=False),
        scratch_types=(
            [pltpu.VMEM((CHUNK,), jnp.int32)] * nbuf
            + [pltpu.VMEM((CHUNK, 16), jnp.float32)]
            + [pltpu.VMEM_SHARED((N, 16), jnp.float32)]
            + [pltpu.SemaphoreType.DMA] * (2 * nbuf)
        ),
    )
    def k(dst_hbm, out_hbm, *scr):
        didx = scr[:nbuf]
        buf = scr[nbuf]
        acc = scr[nbuf + 1]
        dsem = scr[nbuf + 2:nbuf + 2 + nbuf]
        ssem = scr[nbuf + 2 + nbuf:]
        c = lax.axis_index("c")
        s = lax.axis_index("s")
        wid = c * 16 + s
        r0 = s * RPT
        nch = _nch_for(wid)

        _fill_vmem_2d(buf, CHUNK, 16, 0.0)
        _copy_rows(buf, acc, r0, RPT)
        _fill_vmem_2d(buf, CHUNK, 16, 1.0)
        plsc.subcore_barrier()

        def body(g, _):
            for b in range(nbuf):
                j = g * nbuf + b

                @pl.when((j >= nbuf) & (j - nbuf < nch))
                def _retire():
                    pltpu.make_async_copy(buf, acc.at[didx[b]],
                                          ssem[b]).wait()

                @pl.when(j < nch)
                def _load():
                    cid = j * NW + wid
                    pltpu.async_copy(
                        dst_hbm.at[pl.ds(cid * CHUNK, CHUNK)],
                        didx[b], dsem[b])

                bs = (b - slag) % nbuf
                js = j - slag

                @pl.when((js >= 0) & (js < nch))
                def _scatter():
                    pltpu.make_async_copy(
                        dst_hbm.at[pl.ds(0, CHUNK)], didx[bs],
                        dsem[bs]).wait()
                    pltpu.async_copy(buf, acc.at[didx[bs]], ssem[bs],
                                     add=True)
            return 0

        grps = (nch + 2 * nbuf - 1) // nbuf
        lax.fori_loop(0, grps, body, 0)
        plsc.subcore_barrier()
        pltpu.sync_copy(acc.at[pl.ds(r0, RPT), 0:8],
                        out_hbm.at[c, pl.ds(r0, RPT), :])

    return k(dst_flat)


def _sc_propagate(table, src_flat, dst_flat, d, nbuf, slag, untiled):
    cparams = (pltpu.CompilerParams(use_tc_tiling_on_sc=False)
               if untiled else pltpu.CompilerParams())

    @functools.partial(
        pl.kernel,
        out_type=jax.ShapeDtypeStruct((2, NP, d), jnp.float32),
        mesh=plsc.VectorSubcoreMesh(**_MESH),
        compiler_params=cparams,
        scratch_types=(
            [pltpu.VMEM((CHUNK,), jnp.int32)] * nbuf
            + [pltpu.VMEM((CHUNK,), jnp.int32)] * nbuf
            + [pltpu.VMEM((CHUNK, d), jnp.float32)] * nbuf
            + [pltpu.VMEM_SHARED((NP, d), jnp.float32)]
            + [pltpu.SemaphoreType.DMA] * (4 * nbuf)
        ),
    )
    def k(tab_hbm, src_hbm, dst_hbm, out_hbm, *scr):
        sidx = scr[:nbuf]
        didx = scr[nbuf:2 * nbuf]
        rows = scr[2 * nbuf:3 * nbuf]
        acc = scr[3 * nbuf]
        isem = scr[3 * nbuf + 1:4 * nbuf + 1]
        dsem = scr[4 * nbuf + 1:5 * nbuf + 1]
        gsem = scr[5 * nbuf + 1:6 * nbuf + 1]
        ssem = scr[6 * nbuf + 1:]
        c = lax.axis_index("c")
        s = lax.axis_index("s")
        wid = c * 16 + s
        r0 = s * RPT_P
        nch = _nch_for(wid)

        _fill_vmem_2d(rows[0], CHUNK, d, 0.0)
        _copy_rows(rows[0], acc, r0, RPT_P)
        plsc.subcore_barrier()

        def body(g, _):
            for b in range(nbuf):
                j = g * nbuf + b

                @pl.when((j >= nbuf) & (j - nbuf < nch))
                def _retire():
                    pltpu.make_async_copy(rows[b], acc.at[didx[b]],
                                          ssem[b]).wait()

                @pl.when(j < nch)
                def _load():
                    cid = j * NW + wid
                    pltpu.async_copy(
                        src_hbm.at[pl.ds(cid * CHUNK, CHUNK)],
                        sidx[b], isem[b])
                    pltpu.async_copy(
                        dst_hbm.at[pl.ds(cid * CHUNK, CHUNK)],
                        didx[b], dsem[b])

                bg = (b - 1) % nbuf
                jg = j - 1

                @pl.when((jg >= 0) & (jg < nch))
                def _gather():
                    pltpu.make_async_copy(
                        src_hbm.at[pl.ds(0, CHUNK)], sidx[bg],
                        isem[bg]).wait()
                    pltpu.async_copy(tab_hbm.at[sidx[bg]], rows[bg],
                                     gsem[bg])

                bs = (b - slag) % nbuf
                js = j - slag

                @pl.when((js >= 0) & (js < nch))
                def _scatter():
                    pltpu.make_async_copy(
                        dst_hbm.at[pl.ds(0, CHUNK)], didx[bs],
                        dsem[bs]).wait()
                    pltpu.make_async_copy(tab_hbm.at[sidx[bs]], rows[bs],
                                          gsem[bs]).wait()
                    pltpu.async_copy(rows[bs], acc.at[didx[bs]],
                                     ssem[bs], add=True)
            return 0

        grps = (nch + 2 * nbuf - 1) // nbuf
        lax.fori_loop(0, grps, body, 0)
        plsc.subcore_barrier()
        pltpu.sync_copy(acc.at[pl.ds(r0, RPT_P), :],
                        out_hbm.at[c, pl.ds(r0, RPT_P), :])

    return k(table, src_flat, dst_flat)


def _tc_mm(x, w0):
    def body(x_ref, w0_ref, out_ref):
        out_ref[...] = jnp.dot(x_ref[...], w0_ref[...],
                               preferred_element_type=jnp.float32)

    return pl.pallas_call(
        body,
        out_shape=jax.ShapeDtypeStruct((N, 128), jnp.float32),
    )(x, w0)


def _dinv_from(degp_ref):
    deg = degp_ref[0, :, 0:1] + degp_ref[1, :, 0:1] + 1.0
    return lax.rsqrt(deg)


def _tc_scale(mm, degp):
    def body(mm_ref, degp_ref, hs_ref):
        hs_ref[...] = mm_ref[...] * _dinv_from(degp_ref)

    return pl.pallas_call(
        body,
        out_shape=jax.ShapeDtypeStruct((N, 128), jnp.float32),
    )(mm, degp)


def _tc_layer2(part1, hs, degp, w1):
    def body(p_ref, hs_ref, degp_ref, w1_ref, out_ref):
        dinv = _dinv_from(degp_ref)
        acc = p_ref[0, :N] + p_ref[1, :N] + hs_ref[...]
        h1 = jnp.maximum(dinv * acc, 0.0)
        h2 = jnp.dot(h1, w1_ref[...], preferred_element_type=jnp.float32)
        out_ref[...] = h2 * dinv

    return pl.pallas_call(
        body,
        out_shape=jax.ShapeDtypeStruct((N, 64), jnp.float32),
    )(part1, hs, degp, w1)


def _tc_final(part2, hs2, degp):
    def body(p_ref, hs2_ref, degp_ref, out_ref):
        acc = p_ref[0, :N] + p_ref[1, :N] + hs2_ref[...]
        out_ref[...] = _dinv_from(degp_ref) * acc

    return pl.pallas_call(
        body,
        out_shape=jax.ShapeDtypeStruct((N, 64), jnp.float32),
    )(part2, hs2, degp)


def kernel(x, edge_index, W0, W1):
    ei = edge_index.astype(jnp.int32)
    src_flat = ei[0]
    dst_flat = ei[1]
    degp = _sc_degree(dst_flat)
    mm = _tc_mm(x, W0)
    hs = _tc_scale(mm, degp)
    part1 = _sc_propagate(hs, src_flat, dst_flat, 128, nbuf=3, slag=2,
                          untiled=False)
    hs2 = _tc_layer2(part1, hs, degp, W1)
    part2 = _sc_propagate(hs2, src_flat, dst_flat, 64, nbuf=4, slag=2,
                          untiled=True)
    return _tc_final(part2, hs2, degp)


# speedup vs baseline: 1.0010x; 1.0010x over previous
import functools

import jax
import jax.numpy as jnp
from jax import lax
from jax.experimental import pallas as pl
from jax.experimental.pallas import tpu as pltpu
from jax.experimental.pallas import tpu_sc as plsc

N = 10000
E = 320000
CHUNK = 128
NCH = E // CHUNK
NW = 32
RPT = N // 16
NP = 10112
RPT_P = NP // 16

_MESH = dict(core_axis_name="c", subcore_axis_name="s",
             num_cores=2, num_subcores=16)


def _fill_vmem_2d(ref, nrows, ncols, value):
    v = jnp.full((16,), value, jnp.float32)

    def body(r, _):
        for j in range(ncols // 16):
            ref[r, pl.ds(j * 16, 16)] = v
        return 0

    lax.fori_loop(0, nrows, body, 0)


def _copy_rows(src_ref, dst_ref, r0, rpt):
    for p in range(rpt // CHUNK):
        pltpu.sync_copy(src_ref, dst_ref.at[pl.ds(r0 + p * CHUNK, CHUNK), :])
    rem = rpt % CHUNK
    if rem:
        pltpu.sync_copy(
            src_ref.at[pl.ds(0, rem), :],
            dst_ref.at[pl.ds(r0 + (rpt // CHUNK) * CHUNK, rem), :],
        )


def _nch_for(wid):
    return (NCH - wid + NW - 1) // NW


def _sc_degree(dst_flat):
    nbuf = 4
    slag = 2

    @functools.partial(
        pl.kernel,
        out_type=jax.ShapeDtypeStruct((2, N, 8), jnp.float32),
        mesh=plsc.VectorSubcoreMesh(**_MESH),
        compiler_params=pltpu.CompilerParams(use_tc_tiling_on_sc=False),
        scratch_types=(
            [pltpu.VMEM((CHUNK,), jnp.int32)] * nbuf
            + [pltpu.VMEM((CHUNK, 16), jnp.float32)]
            + [pltpu.VMEM_SHARED((N, 16), jnp.float32)]
            + [pltpu.SemaphoreType.DMA] * (2 * nbuf)
        ),
    )
    def k(dst_hbm, out_hbm, *scr):
        didx = scr[:nbuf]
        buf = scr[nbuf]
        acc = scr[nbuf + 1]
        dsem = scr[nbuf + 2:nbuf + 2 + nbuf]
        ssem = scr[nbuf + 2 + nbuf:]
        c = lax.axis_index("c")
        s = lax.axis_index("s")
        wid = c * 16 + s
        r0 = s * RPT
        nch = _nch_for(wid)

        _fill_vmem_2d(buf, CHUNK, 16, 0.0)
        _copy_rows(buf, acc, r0, RPT)
        _fill_vmem_2d(buf, CHUNK, 16, 1.0)
        plsc.subcore_barrier()

        def body(g, _):
            for b in range(nbuf):
                j = g * nbuf + b

                @pl.when((j >= nbuf) & (j - nbuf < nch))
                def _retire():
                    pltpu.make_async_copy(buf, acc.at[didx[b]],
                                          ssem[b]).wait()

                @pl.when(j < nch)
                def _load():
                    cid = j * NW + wid
                    pltpu.async_copy(
                        dst_hbm.at[pl.ds(cid * CHUNK, CHUNK)],
                        didx[b], dsem[b])

                bs = (b - slag) % nbuf
                js = j - slag

                @pl.when((js >= 0) & (js < nch))
                def _scatter():
                    pltpu.make_async_copy(
                        dst_hbm.at[pl.ds(0, CHUNK)], didx[bs],
                        dsem[bs]).wait()
                    pltpu.async_copy(buf, acc.at[didx[bs]], ssem[bs],
                                     add=True)
            return 0

        grps = (nch + 2 * nbuf - 1) // nbuf
        lax.fori_loop(0, grps, body, 0)
        plsc.subcore_barrier()
        pltpu.sync_copy(acc.at[pl.ds(r0, RPT), 0:8],
                        out_hbm.at[c, pl.ds(r0, RPT), :])

    return k(dst_flat)


def _sc_propagate(table, src_flat, dst_flat, d, nbuf, slag, untiled):
    cparams = (pltpu.CompilerParams(use_tc_tiling_on_sc=False)
               if untiled else pltpu.CompilerParams())

    @functools.partial(
        pl.kernel,
        out_type=jax.ShapeDtypeStruct((2, NP, d), jnp.float32),
        mesh=plsc.VectorSubcoreMesh(**_MESH),
        compiler_params=cparams,
        scratch_types=(
            [pltpu.VMEM((CHUNK,), jnp.int32)] * nbuf
            + [pltpu.VMEM((CHUNK,), jnp.int32)] * nbuf
            + [pltpu.VMEM((CHUNK, d), jnp.float32)] * nbuf
            + [pltpu.VMEM_SHARED((NP, d), jnp.float32)]
            + [pltpu.SemaphoreType.DMA] * (4 * nbuf)
        ),
    )
    def k(tab_hbm, src_hbm, dst_hbm, out_hbm, *scr):
        sidx = scr[:nbuf]
        didx = scr[nbuf:2 * nbuf]
        rows = scr[2 * nbuf:3 * nbuf]
        acc = scr[3 * nbuf]
        isem = scr[3 * nbuf + 1:4 * nbuf + 1]
        dsem = scr[4 * nbuf + 1:5 * nbuf + 1]
        gsem = scr[5 * nbuf + 1:6 * nbuf + 1]
        ssem = scr[6 * nbuf + 1:]
        c = lax.axis_index("c")
        s = lax.axis_index("s")
        wid = c * 16 + s
        r0 = s * RPT_P
        nch = _nch_for(wid)

        _fill_vmem_2d(rows[0], CHUNK, d, 0.0)
        _copy_rows(rows[0], acc, r0, RPT_P)
        plsc.subcore_barrier()

        def body(g, _):
            for b in range(nbuf):
                j = g * nbuf + b

                @pl.when((j >= nbuf) & (j - nbuf < nch))
                def _retire():
                    pltpu.make_async_copy(rows[b], acc.at[didx[b]],
                                          ssem[b]).wait()

                @pl.when(j < nch)
                def _load():
                    cid = j * NW + wid
                    pltpu.async_copy(
                        src_hbm.at[pl.ds(cid * CHUNK, CHUNK)],
                        sidx[b], isem[b])
                    pltpu.async_copy(
                        dst_hbm.at[pl.ds(cid * CHUNK, CHUNK)],
                        didx[b], dsem[b])

                bg = (b - 1) % nbuf
                jg = j - 1

                @pl.when((jg >= 0) & (jg < nch))
                def _gather():
                    pltpu.make_async_copy(
                        src_hbm.at[pl.ds(0, CHUNK)], sidx[bg],
                        isem[bg]).wait()
                    pltpu.async_copy(tab_hbm.at[sidx[bg]], rows[bg],
                                     gsem[bg])

                bs = (b - slag) % nbuf
                js = j - slag

                @pl.when((js >= 0) & (js < nch))
                def _scatter():
                    pltpu.make_async_copy(
                        dst_hbm.at[pl.ds(0, CHUNK)], didx[bs],
                        dsem[bs]).wait()
                    pltpu.make_async_copy(tab_hbm.at[sidx[bs]], rows[bs],
                                          gsem[bs]).wait()
                    pltpu.async_copy(rows[bs], acc.at[didx[bs]],
                                     ssem[bs], add=True)
            return 0

        grps = (nch + 2 * nbuf - 1) // nbuf
        lax.fori_loop(0, grps, body, 0)
        plsc.subcore_barrier()
        pltpu.sync_copy(acc.at[pl.ds(r0, RPT_P), :],
                        out_hbm.at[c, pl.ds(r0, RPT_P), :])

    return k(table, src_flat, dst_flat)


def _dinv_from(degp_ref):
    deg = degp_ref[0, :, 0:1] + degp_ref[1, :, 0:1] + 1.0
    return lax.rsqrt(deg)


def _tc_layer1(x, w0, degp):
    def body(x_ref, w0_ref, degp_ref, hs_ref):
        h = jnp.dot(x_ref[...], w0_ref[...],
                    preferred_element_type=jnp.float32)
        hs_ref[...] = h * _dinv_from(degp_ref)

    return pl.pallas_call(
        body,
        out_shape=jax.ShapeDtypeStruct((N, 128), jnp.float32),
    )(x, w0, degp)


def _tc_layer2(part1, hs, degp, w1):
    def body(p_ref, hs_ref, degp_ref, w1_ref, out_ref):
        dinv = _dinv_from(degp_ref)
        acc = p_ref[0, :N] + p_ref[1, :N] + hs_ref[...]
        h1 = jnp.maximum(dinv * acc, 0.0)
        h2 = jnp.dot(h1, w1_ref[...], preferred_element_type=jnp.float32)
        out_ref[...] = h2 * dinv

    return pl.pallas_call(
        body,
        out_shape=jax.ShapeDtypeStruct((N, 64), jnp.float32),
    )(part1, hs, degp, w1)


def _tc_final(part2, hs2, degp):
    def body(p_ref, hs2_ref, degp_ref, out_ref):
        acc = p_ref[0, :N] + p_ref[1, :N] + hs2_ref[...]
        out_ref[...] = _dinv_from(degp_ref) * acc

    return pl.pallas_call(
        body,
        out_shape=jax.ShapeDtypeStruct((N, 64), jnp.float32),
    )(part2, hs2, degp)


def kernel(x, edge_index, W0, W1):
    ei = edge_index.astype(jnp.int32)
    src_flat = ei[0]
    dst_flat = ei[1]
    degp = _sc_degree(dst_flat)
    hs = _tc_layer1(x, W0, degp)
    part1 = _sc_propagate(hs, src_flat, dst_flat, 128, nbuf=3, slag=2,
                          untiled=False)
    hs2 = _tc_layer2(part1, hs, degp, W1)
    part2 = _sc_propagate(hs2, src_flat, dst_flat, 64, nbuf=4, slag=2,
                          untiled=True)
    return _tc_final(part2, hs2, degp)


# speedup vs baseline: 1.0015x; 1.0005x over previous
import functools

import jax
import jax.numpy as jnp
from jax import lax
from jax.experimental import pallas as pl
from jax.experimental.pallas import tpu as pltpu
from jax.experimental.pallas import tpu_sc as plsc

N = 10000
E = 320000
CHUNK = 128
NCH = E // CHUNK
NW = 32
RPT = N // 16
NP = 10112
RPT_P = NP // 16

_MESH = dict(core_axis_name="c", subcore_axis_name="s",
             num_cores=2, num_subcores=16)


def _fill_vmem_2d(ref, nrows, ncols, value):
    v = jnp.full((16,), value, jnp.float32)

    def body(r, _):
        for j in range(ncols // 16):
            ref[r, pl.ds(j * 16, 16)] = v
        return 0

    lax.fori_loop(0, nrows, body, 0)


def _copy_rows(src_ref, dst_ref, r0, rpt):
    for p in range(rpt // CHUNK):
        pltpu.sync_copy(src_ref, dst_ref.at[pl.ds(r0 + p * CHUNK, CHUNK), :])
    rem = rpt % CHUNK
    if rem:
        pltpu.sync_copy(
            src_ref.at[pl.ds(0, rem), :],
            dst_ref.at[pl.ds(r0 + (rpt // CHUNK) * CHUNK, rem), :],
        )


def _nch_for(wid):
    return (NCH - wid + NW - 1) // NW


def _sc_degree(dst_flat):
    nbuf = 4
    slag = 2

    @functools.partial(
        pl.kernel,
        out_type=jax.ShapeDtypeStruct((2, N, 8), jnp.float32),
        mesh=plsc.VectorSubcoreMesh(**_MESH),
        compiler_params=pltpu.CompilerParams(use_tc_tiling_on_sc=False),
        scratch_types=(
            [pltpu.VMEM((CHUNK,), jnp.int32)] * nbuf
            + [pltpu.VMEM((CHUNK, 16), jnp.float32)]
            + [pltpu.VMEM_SHARED((N, 16), jnp.float32)]
            + [pltpu.SemaphoreType.DMA] * (2 * nbuf)
        ),
    )
    def k(dst_hbm, out_hbm, *scr):
        didx = scr[:nbuf]
        buf = scr[nbuf]
        acc = scr[nbuf + 1]
        dsem = scr[nbuf + 2:nbuf + 2 + nbuf]
        ssem = scr[nbuf + 2 + nbuf:]
        c = lax.axis_index("c")
        s = lax.axis_index("s")
        wid = c * 16 + s
        r0 = s * RPT
        nch = _nch_for(wid)

        _fill_vmem_2d(buf, CHUNK, 16, 0.0)
        _copy_rows(buf, acc, r0, RPT)
        _fill_vmem_2d(buf, CHUNK, 16, 1.0)
        plsc.subcore_barrier()

        def body(g, _):
            for b in range(nbuf):
                j = g * nbuf + b

                @pl.when((j >= nbuf) & (j - nbuf < nch))
                def _retire():
                    pltpu.make_async_copy(buf, acc.at[didx[b]],
                                          ssem[b]).wait()

                @pl.when(j < nch)
                def _load():
                    cid = j * NW + wid
                    pltpu.async_copy(
                        dst_hbm.at[pl.ds(cid * CHUNK, CHUNK)],
                        didx[b], dsem[b])

                bs = (b - slag) % nbuf
                js = j - slag

                @pl.when((js >= 0) & (js < nch))
                def _scatter():
                    pltpu.make_async_copy(
                        dst_hbm.at[pl.ds(0, CHUNK)], didx[bs],
                        dsem[bs]).wait()
                    pltpu.async_copy(buf, acc.at[didx[bs]], ssem[bs],
                                     add=True)
            return 0

        grps = (nch + 2 * nbuf - 1) // nbuf
        lax.fori_loop(0, grps, body, 0)
        plsc.subcore_barrier()
        pltpu.sync_copy(acc.at[pl.ds(r0, RPT), 0:8],
                        out_hbm.at[c, pl.ds(r0, RPT), :])

    return k(dst_flat)


def _sc_propagate(table, src_flat, dst_flat, d, nbuf, slag, untiled):
    cparams = (pltpu.CompilerParams(use_tc_tiling_on_sc=False)
               if untiled else pltpu.CompilerParams())

    @functools.partial(
        pl.kernel,
        out_type=jax.ShapeDtypeStruct((2, NP, d), jnp.float32),
        mesh=plsc.VectorSubcoreMesh(**_MESH),
        compiler_params=cparams,
        scratch_types=(
            [pltpu.VMEM((CHUNK,), jnp.int32)] * nbuf
            + [pltpu.VMEM((CHUNK,), jnp.int32)] * nbuf
            + [pltpu.VMEM((CHUNK, d), jnp.float32)] * nbuf
            + [pltpu.VMEM_SHARED((NP, d), jnp.float32)]
            + [pltpu.SemaphoreType.DMA] * (4 * nbuf)
        ),
    )
    def k(tab_hbm, src_hbm, dst_hbm, out_hbm, *scr):
        sidx = scr[:nbuf]
        didx = scr[nbuf:2 * nbuf]
        rows = scr[2 * nbuf:3 * nbuf]
        acc = scr[3 * nbuf]
        isem = scr[3 * nbuf + 1:4 * nbuf + 1]
        dsem = scr[4 * nbuf + 1:5 * nbuf + 1]
        gsem = scr[5 * nbuf + 1:6 * nbuf + 1]
        ssem = scr[6 * nbuf + 1:]
        c = lax.axis_index("c")
        s = lax.axis_index("s")
        wid = c * 16 + s
        r0 = s * RPT_P
        nch = _nch_for(wid)

        _fill_vmem_2d(rows[0], CHUNK, d, 0.0)
        _copy_rows(rows[0], acc, r0, RPT_P)
        plsc.subcore_barrier()

        def body(g, _):
            for b in range(nbuf):
                j = g * nbuf + b

                @pl.when((j >= nbuf) & (j - nbuf < nch))
                def _retire():
                    pltpu.make_async_copy(rows[b], acc.at[didx[b]],
                                          ssem[b]).wait()

                @pl.when(j < nch)
                def _load():
                    cid = j * NW + wid
                    pltpu.async_copy(
                        src_hbm.at[pl.ds(cid * CHUNK, CHUNK)],
                        sidx[b], isem[b])
                    pltpu.async_copy(
                        dst_hbm.at[pl.ds(cid * CHUNK, CHUNK)],
                        didx[b], dsem[b])

                bg = (b - 1) % nbuf
                jg = j - 1

                @pl.when((jg >= 0) & (jg < nch))
                def _gather():
                    pltpu.make_async_copy(
                        src_hbm.at[pl.ds(0, CHUNK)], sidx[bg],
                        isem[bg]).wait()
                    pltpu.async_copy(tab_hbm.at[sidx[bg]], rows[bg],
                                     gsem[bg])

                bs = (b - slag) % nbuf
                js = j - slag

                @pl.when((js >= 0) & (js < nch))
                def _scatter():
                    pltpu.make_async_copy(
                        dst_hbm.at[pl.ds(0, CHUNK)], didx[bs],
                        dsem[bs]).wait()
                    pltpu.make_async_copy(tab_hbm.at[sidx[bs]], rows[bs],
                                          gsem[bs]).wait()
                    pltpu.async_copy(rows[bs], acc.at[didx[bs]],
                                     ssem[bs], add=True)
            return 0

        grps = (nch + 2 * nbuf - 1) // nbuf
        lax.fori_loop(0, grps, body, 0)
        plsc.subcore_barrier()
        pltpu.sync_copy(acc.at[pl.ds(r0, RPT_P), :],
                        out_hbm.at[c, pl.ds(r0, RPT_P), :])

    return k(table, src_flat, dst_flat)


def _dinv_from(degp_ref):
    deg = degp_ref[0, :, 0:1] + degp_ref[1, :, 0:1] + 1.0
    return lax.rsqrt(deg)


def _tc_layer1(x, w0, degp):
    def body(x_ref, w0_ref, degp_ref, hs_ref):
        h = jnp.dot(x_ref[...], w0_ref[...],
                    preferred_element_type=jnp.float32)
        hs_ref[...] = h * _dinv_from(degp_ref)

    return pl.pallas_call(
        body,
        out_shape=jax.ShapeDtypeStruct((N, 128), jnp.float32),
    )(x, w0, degp)


def _tc_layer2(part1, hs, degp, w1):
    def body(p_ref, hs_ref, degp_ref, w1_ref, out_ref):
        dinv = _dinv_from(degp_ref)
        acc = p_ref[0, :N] + p_ref[1, :N] + hs_ref[...]
        h1 = jnp.maximum(dinv * acc, 0.0)
        h2 = jnp.dot(h1, w1_ref[...], preferred_element_type=jnp.float32)
        out_ref[...] = h2 * dinv

    return pl.pallas_call(
        body,
        out_shape=jax.ShapeDtypeStruct((N, 64), jnp.float32),
    )(part1, hs, degp, w1)


def _tc_final(part2, hs2, degp):
    def body(p_ref, hs2_ref, degp_ref, out_ref):
        acc = p_ref[0, :N] + p_ref[1, :N] + hs2_ref[...]
        out_ref[...] = _dinv_from(degp_ref) * acc

    return pl.pallas_call(
        body,
        out_shape=jax.ShapeDtypeStruct((N, 64), jnp.float32),
    )(part2, hs2, degp)


def kernel(x, edge_index, W0, W1):
    ei = edge_index.astype(jnp.int32)
    src_flat = ei[0]
    dst_flat = ei[1]
    degp = _sc_degree(dst_flat)
    hs = _tc_layer1(x, W0, degp)
    part1 = _sc_propagate(hs, src_flat, dst_flat, 128, nbuf=3, slag=2,
                          untiled=True)
    hs2 = _tc_layer2(part1, hs, degp, W1)
    part2 = _sc_propagate(hs2, src_flat, dst_flat, 64, nbuf=4, slag=2,
                          untiled=True)
    return _tc_final(part2, hs2, degp)
